# baseline (speedup 1.0000x reference)
import jax
import jax.numpy as jnp
from jax.experimental import pallas as pl
from jax.experimental.pallas import tpu as pltpu

N = 10000
NFEAT = 128
NHID = 16
NCLASS = 8

TI = 400
NI = N // TI


def _layer1_body(adj_ref, x_ref, w1_ref, b1_ref, h_ref, s1_ref):
    @pl.when(pl.program_id(0) == 0)
    def _():
        s1_ref[...] = jnp.dot(x_ref[...], w1_ref[...],
                              preferred_element_type=jnp.float32)

    acc = jnp.dot(adj_ref[...], s1_ref[...],
                  preferred_element_type=jnp.float32)
    h_ref[...] = jnp.maximum(acc + b1_ref[...], 0.0)


def _layer2_body(adj_ref, h_ref, w2_ref, b2_ref, o_ref, s2_ref):
    @pl.when(pl.program_id(0) == 0)
    def _():
        s2_ref[...] = jnp.dot(h_ref[...], w2_ref[...],
                              preferred_element_type=jnp.float32)

    z = jnp.dot(adj_ref[...], s2_ref[...],
                preferred_element_type=jnp.float32) + b2_ref[...]
    m = jnp.max(z, axis=1, keepdims=True)
    lse = jnp.log(jnp.sum(jnp.exp(z - m), axis=1, keepdims=True)) + m
    o_ref[...] = z - lse


def kernel(x, adj, W1, b1, W2, b2):
    b1r = b1.reshape(1, NHID)
    b2r = b2.reshape(1, NCLASS)

    h = pl.pallas_call(
        _layer1_body,
        grid=(NI,),
        in_specs=[
            pl.BlockSpec((TI, N), lambda i: (i, 0)),
            pl.BlockSpec((N, NFEAT), lambda i: (0, 0)),
            pl.BlockSpec((NFEAT, NHID), lambda i: (0, 0)),
            pl.BlockSpec((1, NHID), lambda i: (0, 0)),
        ],
        out_specs=pl.BlockSpec((TI, NHID), lambda i: (i, 0)),
        out_shape=jax.ShapeDtypeStruct((N, NHID), jnp.float32),
        scratch_shapes=[pltpu.VMEM((N, NHID), jnp.float32)],
        compiler_params=pltpu.CompilerParams(
            dimension_semantics=("arbitrary",)),
    )(adj, x, W1, b1r)

    out = pl.pallas_call(
        _layer2_body,
        grid=(NI,),
        in_specs=[
            pl.BlockSpec((TI, N), lambda i: (i, 0)),
            pl.BlockSpec((N, NHID), lambda i: (0, 0)),
            pl.BlockSpec((NHID, NCLASS), lambda i: (0, 0)),
            pl.BlockSpec((1, NCLASS), lambda i: (0, 0)),
        ],
        out_specs=pl.BlockSpec((TI, NCLASS), lambda i: (i, 0)),
        out_shape=jax.ShapeDtypeStruct((N, NCLASS), jnp.float32),
        scratch_shapes=[pltpu.VMEM((N, NCLASS), jnp.float32)],
        compiler_params=pltpu.CompilerParams(
            dimension_semantics=("arbitrary",)),
    )(adj, h, W2, b2r)

    return out
